# baseline (speedup 1.0000x reference)
import functools

import jax
import jax.numpy as jnp
from jax import lax
from jax.experimental import pallas as pl
from jax.experimental.pallas import tpu as pltpu
from jax.experimental.pallas import tpu_sc as plsc

NC = 2
NS = 16
L = 16
CHUNK = 128
NBUF = 3
ATAB = 208


def _sc_aggregate(xb0, xb1, src, dst, rel, alpha_flat):
    n, dw = xb0.shape
    dh = 2 * dw
    e = src.shape[0]
    assert e % CHUNK == 0
    num_chunks = e // CHUNK
    full_rounds = num_chunks // NS
    extra = num_chunks - full_rounds * NS
    assert full_rounds % NBUF == 0
    outer = full_rounds // NBUF
    assert n % NS == 0
    zero_per_sub = n // NS
    out_per_sub = -(-n // (NS * 8)) * 8
    n_pad = out_per_sub * NS
    last_rows = n - out_per_sub * (NS - 1)
    assert last_rows > 0 and last_rows % 8 == 0

    mesh = plsc.VectorSubcoreMesh(
        core_axis_name="c", subcore_axis_name="s", num_cores=NC, num_subcores=NS
    )

    @functools.partial(
        pl.kernel,
        out_type=jax.ShapeDtypeStruct((NC, n_pad, dh), jnp.float32),
        mesh=mesh,
        scratch_types=[
            [pltpu.VMEM((CHUNK,), jnp.int32) for _ in range(NBUF)],
            [pltpu.VMEM((CHUNK,), jnp.int32) for _ in range(NBUF)],
            [pltpu.VMEM((CHUNK,), jnp.int32) for _ in range(NBUF)],
            pltpu.VMEM((CHUNK,), jnp.float32),
            [pltpu.VMEM((CHUNK, dw), jnp.int32) for _ in range(NBUF)],
            [pltpu.VMEM((CHUNK, dw), jnp.int32) for _ in range(NBUF)],
            [pltpu.VMEM((CHUNK, dh), jnp.float32) for _ in range(NBUF)],
            [pltpu.VMEM((CHUNK, dh), jnp.float32) for _ in range(NBUF)],
            pltpu.VMEM((ATAB,), jnp.float32),
            pltpu.VMEM_SHARED((n, dh), jnp.float32),
            [pltpu.SemaphoreType.DMA for _ in range(3)],
            [pltpu.SemaphoreType.DMA for _ in range(NBUF)],
            [pltpu.SemaphoreType.DMA for _ in range(NBUF)],
            [pltpu.SemaphoreType.DMA for _ in range(NBUF)],
            [pltpu.SemaphoreType.DMA for _ in range(NBUF)],
        ],
        compiler_params=pltpu.CompilerParams(
            needs_layout_passes=False, use_tc_tiling_on_sc=False
        ),
    )
    def agg(x0_hbm, x1_hbm, src_hbm, dst_hbm, rel_hbm, alpha_hbm, out_hbm,
            sv, dv, rl, alp_v, rind, rins, routd, routs, alpha_v, acc,
            isem, gsem_d, gsem_s, ssem_d, ssem_s):
        cid_ax = lax.axis_index("c")
        sid = lax.axis_index("s")

        pltpu.sync_copy(alpha_hbm, alpha_v)

        zero16 = jnp.zeros((L,), jnp.float32)

        def zrow(i, carry):
            for cc in range(dh // L):
                routd[0][i, pl.ds(cc * L, L)] = zero16
            return carry

        lax.fori_loop(0, CHUNK, zrow, 0)
        zbase = sid * zero_per_sub
        n_full = zero_per_sub // CHUNK
        tail = zero_per_sub - n_full * CHUNK
        for i in range(n_full):
            pltpu.sync_copy(routd[0], acc.at[pl.ds(zbase + i * CHUNK, CHUNK)])
        if tail:
            pltpu.sync_copy(routd[0].at[pl.ds(0, tail)],
                            acc.at[pl.ds(zbase + n_full * CHUNK, tail)])
        plsc.subcore_barrier()

        def gather_into(b, cid):
            base = cid * CHUNK
            a1 = pltpu.async_copy(src_hbm.at[pl.ds(base, CHUNK)], sv[b], isem[0])
            a2 = pltpu.async_copy(dst_hbm.at[pl.ds(base, CHUNK)], dv[b], isem[1])
            a3 = pltpu.async_copy(rel_hbm.at[pl.ds(base, CHUNK)], rl[b], isem[2])
            a1.wait()
            a2.wait()
            a3.wait()

            @pl.when(cid_ax == 0)
            def _():
                pltpu.async_copy(x0_hbm.at[dv[b]], rind[b], gsem_d[b])
                pltpu.async_copy(x0_hbm.at[sv[b]], rins[b], gsem_s[b])

            @pl.when(cid_ax == 1)
            def _():
                pltpu.async_copy(x1_hbm.at[dv[b]], rind[b], gsem_d[b])
                pltpu.async_copy(x1_hbm.at[sv[b]], rins[b], gsem_s[b])

        def wait_gathers(b):
            pltpu.make_async_copy(x0_hbm.at[dv[b]], rind[b], gsem_d[b]).wait()
            pltpu.make_async_copy(x0_hbm.at[sv[b]], rins[b], gsem_s[b]).wait()

        def scale_and_scatter(b):
            for k8 in range(CHUNK // L):
                r16 = rl[b][pl.ds(k8 * L, L)]
                alp_v[pl.ds(k8 * L, L)] = plsc.load_gather(alpha_v, [r16])

            def edge_group(k, carry):
                a16 = alp_v[pl.ds(k * L, L)]
                for j in range(L):
                    a = jnp.full((L,), a16[j], jnp.float32)
                    row = k * L + j
                    for cc in range(dh // (2 * L)):
                        vwd = rind[b][row, pl.ds(cc * L, L)]
                        vws = rins[b][row, pl.ds(cc * L, L)]
                        routd[b][row, pl.ds(cc * 2 * L, L)] = (
                            plsc.bitcast(vwd << 16, jnp.float32) * a)
                        routd[b][row, pl.ds(cc * 2 * L + L, L)] = (
                            plsc.bitcast(vwd & jnp.int32(-65536), jnp.float32) * a)
                        routs[b][row, pl.ds(cc * 2 * L, L)] = (
                            plsc.bitcast(vws << 16, jnp.float32) * a)
                        routs[b][row, pl.ds(cc * 2 * L + L, L)] = (
                            plsc.bitcast(vws & jnp.int32(-65536), jnp.float32) * a)
                return carry

            lax.fori_loop(0, CHUNK // L, edge_group, 0)
            pltpu.async_copy(routd[b], acc.at[sv[b]], ssem_d[b], add=True)
            pltpu.async_copy(routs[b], acc.at[dv[b]], ssem_s[b], add=True)

        def wait_scatters(b):
            pltpu.make_async_copy(routd[b], acc.at[sv[b]], ssem_d[b]).wait()
            pltpu.make_async_copy(routs[b], acc.at[dv[b]], ssem_s[b]).wait()

        if extra:
            @pl.when(sid < extra)
            def _():
                gather_into(0, full_rounds * NS + sid)
                wait_gathers(0)
                scale_and_scatter(0)
                wait_scatters(0)

        gather_into(0, 0 * NS + sid)
        gather_into(1, 1 * NS + sid)

        def body(tt, carry):
            for b in range(NBUF):
                t = NBUF * tt + b
                wait_gathers(b)
                pr = (b + 2) % NBUF
                if b == 0:
                    @pl.when(tt >= 1)
                    def _():
                        wait_scatters(pr)
                    gather_into(pr, (t + 2) * NS + sid)
                else:
                    @pl.when(tt < outer - 1)
                    def _():
                        wait_scatters(pr)
                        gather_into(pr, (t + 2) * NS + sid)
                scale_and_scatter(b)
            return carry

        lax.fori_loop(0, outer, body, 0)
        for b in range(NBUF):
            wait_scatters(b)

        plsc.subcore_barrier()
        obase = sid * out_per_sub

        @pl.when(sid < NS - 1)
        def _():
            pltpu.sync_copy(acc.at[pl.ds(obase, out_per_sub)],
                            out_hbm.at[cid_ax, pl.ds(obase, out_per_sub)])

        @pl.when(sid == NS - 1)
        def _():
            lbase = (NS - 1) * out_per_sub
            pltpu.sync_copy(acc.at[pl.ds(lbase, last_rows)],
                            out_hbm.at[cid_ax, pl.ds(lbase, last_rows)])

    return agg(xb0, xb1, src, dst, rel, alpha_flat)


def _tc_combine_matmul(partial, w, bias2d, n):
    dh = partial.shape[2]
    d = w.shape[0]
    blk = 400
    assert n % blk == 0 and 2 * dh == d

    def body(p0_ref, p1_ref, w_ref, b_ref, o_ref):
        o_ref[...] = (
            jnp.dot(p0_ref[0], w_ref[0:dh, :],
                    preferred_element_type=jnp.float32)
            + jnp.dot(p1_ref[0], w_ref[dh:, :],
                      preferred_element_type=jnp.float32)
            + b_ref[...]
        )

    return pl.pallas_call(
        body,
        grid=(n // blk,),
        in_specs=[
            pl.BlockSpec((1, blk, dh), lambda i: (0, i, 0)),
            pl.BlockSpec((1, blk, dh), lambda i: (1, i, 0)),
            pl.BlockSpec((d, w.shape[1]), lambda i: (0, 0)),
            pl.BlockSpec((1, w.shape[1]), lambda i: (0, 0)),
        ],
        out_specs=pl.BlockSpec((blk, w.shape[1]), lambda i: (i, 0)),
        out_shape=jax.ShapeDtypeStruct((n, w.shape[1]), jnp.float32),
    )(partial, partial, w, bias2d)


def _pack_half(xh):
    n, dh = xh.shape
    xb = (
        xh.reshape(n, dh // 32, 2, 16)
        .transpose(0, 1, 3, 2)
        .reshape(n, dh // 2, 2)
        .astype(jnp.bfloat16)
    )
    return lax.bitcast_convert_type(xb, jnp.int32)


def kernel(input, edge_index, rel_type, n_nodes, W, alpha_table, bias):
    x = input
    n, d = x.shape
    xb0 = _pack_half(x[:, : d // 2])
    xb1 = _pack_half(x[:, d // 2:])
    alpha_flat = jnp.pad(alpha_table[:, 0], (0, ATAB - alpha_table.shape[0]))
    partial = _sc_aggregate(xb0, xb1, edge_index[0], edge_index[1], rel_type,
                            alpha_flat)
    return _tc_combine_matmul(partial, W, bias.reshape(1, -1), n)


# speedup vs baseline: 1.4447x; 1.4447x over previous
import functools

import jax
import jax.numpy as jnp
from jax import lax
from jax.experimental import pallas as pl
from jax.experimental.pallas import tpu as pltpu
from jax.experimental.pallas import tpu_sc as plsc

NC = 2
NS = 16
L = 16
CHUNK = 64
NBUF = 3
ATAB = 208


def _sc_aggregate(x, src, dst, rel, alpha_flat):
    n, d = x.shape
    e = src.shape[0]
    assert e % CHUNK == 0
    num_chunks = e // CHUNK
    nw = NC * NS
    full_rounds = num_chunks // nw
    extra = num_chunks - full_rounds * nw
    assert full_rounds % NBUF == 0
    outer = full_rounds // NBUF
    assert n % NS == 0
    zero_per_sub = n // NS
    out_per_sub = -(-n // (NS * 8)) * 8
    n_pad = out_per_sub * NS
    last_rows = n - out_per_sub * (NS - 1)
    assert last_rows > 0 and last_rows % 8 == 0

    mesh = plsc.VectorSubcoreMesh(
        core_axis_name="c", subcore_axis_name="s", num_cores=NC, num_subcores=NS
    )

    @functools.partial(
        pl.kernel,
        out_type=jax.ShapeDtypeStruct((NC, n_pad, d), jnp.float32),
        mesh=mesh,
        scratch_types=[
            [pltpu.VMEM((CHUNK,), jnp.int32) for _ in range(NBUF)],
            [pltpu.VMEM((CHUNK,), jnp.int32) for _ in range(NBUF)],
            [pltpu.VMEM((CHUNK,), jnp.int32) for _ in range(NBUF)],
            pltpu.VMEM((CHUNK,), jnp.float32),
            [pltpu.VMEM((CHUNK, d), jnp.float32) for _ in range(NBUF)],
            [pltpu.VMEM((CHUNK, d), jnp.float32) for _ in range(NBUF)],
            pltpu.VMEM((ATAB,), jnp.float32),
            pltpu.VMEM_SHARED((n, d), jnp.float32),
            [pltpu.SemaphoreType.DMA for _ in range(3)],
            [pltpu.SemaphoreType.DMA for _ in range(NBUF)],
            [pltpu.SemaphoreType.DMA for _ in range(NBUF)],
            [pltpu.SemaphoreType.DMA for _ in range(NBUF)],
            [pltpu.SemaphoreType.DMA for _ in range(NBUF)],
        ],
        compiler_params=pltpu.CompilerParams(needs_layout_passes=False),
    )
    def agg(x_hbm, src_hbm, dst_hbm, rel_hbm, alpha_hbm, out_hbm,
            sv, dv, rl, alp_v, rd, rs, alpha_v, acc,
            isem, gsem_d, gsem_s, ssem_d, ssem_s):
        cid_ax = lax.axis_index("c")
        sid = lax.axis_index("s")
        wid = sid * NC + cid_ax

        pltpu.sync_copy(alpha_hbm, alpha_v)

        zero16 = jnp.zeros((L,), jnp.float32)

        def zrow(i, carry):
            for cc in range(d // L):
                rd[0][i, pl.ds(cc * L, L)] = zero16
            return carry

        lax.fori_loop(0, CHUNK, zrow, 0)
        zbase = sid * zero_per_sub
        n_full = zero_per_sub // CHUNK
        tail = zero_per_sub - n_full * CHUNK
        for i in range(n_full):
            pltpu.sync_copy(rd[0], acc.at[pl.ds(zbase + i * CHUNK, CHUNK)])
        if tail:
            pltpu.sync_copy(rd[0].at[pl.ds(0, tail)],
                            acc.at[pl.ds(zbase + n_full * CHUNK, tail)])
        plsc.subcore_barrier()

        def gather_into(b, cid):
            base = cid * CHUNK
            a1 = pltpu.async_copy(src_hbm.at[pl.ds(base, CHUNK)], sv[b], isem[0])
            a2 = pltpu.async_copy(dst_hbm.at[pl.ds(base, CHUNK)], dv[b], isem[1])
            a3 = pltpu.async_copy(rel_hbm.at[pl.ds(base, CHUNK)], rl[b], isem[2])
            a1.wait()
            a2.wait()
            a3.wait()
            pltpu.async_copy(x_hbm.at[dv[b]], rd[b], gsem_d[b])
            pltpu.async_copy(x_hbm.at[sv[b]], rs[b], gsem_s[b])

        def wait_gathers(b):
            pltpu.make_async_copy(x_hbm.at[dv[b]], rd[b], gsem_d[b]).wait()
            pltpu.make_async_copy(x_hbm.at[sv[b]], rs[b], gsem_s[b]).wait()

        def scale_and_scatter(b):
            for k8 in range(CHUNK // L):
                r16 = rl[b][pl.ds(k8 * L, L)]
                alp_v[pl.ds(k8 * L, L)] = plsc.load_gather(alpha_v, [r16])

            def edge_group(k, carry):
                a16 = alp_v[pl.ds(k * L, L)]
                for j in range(L):
                    a = jnp.full((L,), a16[j], jnp.float32)
                    row = k * L + j
                    for cc in range(d // L):
                        sl = pl.ds(cc * L, L)
                        rd[b][row, sl] = rd[b][row, sl] * a
                        rs[b][row, sl] = rs[b][row, sl] * a
                return carry

            lax.fori_loop(0, CHUNK // L, edge_group, 0)
            pltpu.async_copy(rd[b], acc.at[sv[b]], ssem_d[b], add=True)
            pltpu.async_copy(rs[b], acc.at[dv[b]], ssem_s[b], add=True)

        def wait_scatters(b):
            pltpu.make_async_copy(rd[b], acc.at[sv[b]], ssem_d[b]).wait()
            pltpu.make_async_copy(rs[b], acc.at[dv[b]], ssem_s[b]).wait()

        if extra:
            @pl.when(wid < extra)
            def _():
                gather_into(0, full_rounds * nw + wid)
                wait_gathers(0)
                scale_and_scatter(0)
                wait_scatters(0)

        gather_into(0, 0 * nw + wid)
        gather_into(1, 1 * nw + wid)

        def body(tt, carry):
            for b in range(NBUF):
                t = NBUF * tt + b
                wait_gathers(b)
                pr = (b + 2) % NBUF
                if b == 0:
                    @pl.when(tt >= 1)
                    def _():
                        wait_scatters(pr)
                    gather_into(pr, (t + 2) * nw + wid)
                else:
                    @pl.when(tt < outer - 1)
                    def _():
                        wait_scatters(pr)
                        gather_into(pr, (t + 2) * nw + wid)
                scale_and_scatter(b)
            return carry

        lax.fori_loop(0, outer, body, 0)
        for b in range(NBUF):
            wait_scatters(b)

        plsc.subcore_barrier()
        obase = sid * out_per_sub

        @pl.when(sid < NS - 1)
        def _():
            pltpu.sync_copy(acc.at[pl.ds(obase, out_per_sub)],
                            out_hbm.at[cid_ax, pl.ds(obase, out_per_sub)])

        @pl.when(sid == NS - 1)
        def _():
            lbase = (NS - 1) * out_per_sub
            pltpu.sync_copy(acc.at[pl.ds(lbase, last_rows)],
                            out_hbm.at[cid_ax, pl.ds(lbase, last_rows)])

    return agg(x, src, dst, rel, alpha_flat)


def _tc_combine_matmul(partial, w, bias2d, n):
    d = partial.shape[2]
    blk = 400
    assert n % blk == 0

    def body(p0_ref, p1_ref, w_ref, b_ref, o_ref):
        sup = p0_ref[0] + p1_ref[0]
        o_ref[...] = (
            jnp.dot(sup, w_ref[...], preferred_element_type=jnp.float32)
            + b_ref[...]
        )

    return pl.pallas_call(
        body,
        grid=(n // blk,),
        in_specs=[
            pl.BlockSpec((1, blk, d), lambda i: (0, i, 0)),
            pl.BlockSpec((1, blk, d), lambda i: (1, i, 0)),
            pl.BlockSpec((d, w.shape[1]), lambda i: (0, 0)),
            pl.BlockSpec((1, w.shape[1]), lambda i: (0, 0)),
        ],
        out_specs=pl.BlockSpec((blk, w.shape[1]), lambda i: (i, 0)),
        out_shape=jax.ShapeDtypeStruct((n, w.shape[1]), jnp.float32),
    )(partial, partial, w, bias2d)


def kernel(input, edge_index, rel_type, n_nodes, W, alpha_table, bias):
    x = input
    alpha_flat = jnp.pad(alpha_table[:, 0], (0, ATAB - alpha_table.shape[0]))
    partial = _sc_aggregate(x, edge_index[0], edge_index[1], rel_type, alpha_flat)
    return _tc_combine_matmul(partial, W, bias.reshape(1, -1), x.shape[0])
